# baseline (speedup 1.0000x reference)
import functools

import jax
import jax.numpy as jnp
from jax import lax
from jax.experimental import pallas as pl
from jax.experimental.pallas import tpu as pltpu
from jax.experimental.pallas import tpu_sc as plsc

_B, _S, _W = 4, 2048, 4096
_R = _B * _S
_RSC = 3072
_NW = 32
_RPW = _RSC // _NW
_CR = 8
_CC = 2048
_L = 16


@functools.partial(
    pl.kernel,
    out_type=jax.ShapeDtypeStruct((_RSC, 2 * _W), jnp.float32),
    mesh=plsc.VectorSubcoreMesh(core_axis_name="c", subcore_axis_name="s"),
    scratch_types=[
        pltpu.VMEM((_CR, _CC), jnp.float32),
        pltpu.VMEM((_CR, _CC), jnp.float32),
        pltpu.VMEM((_CR, 2 * _CC), jnp.float32),
        pltpu.VMEM((_CR, 2 * _CC), jnp.float32),
        pltpu.SemaphoreType.DMA,
        pltpu.SemaphoreType.DMA,
        pltpu.SemaphoreType.DMA,
        pltpu.SemaphoreType.DMA,
    ],
    compiler_params=pltpu.CompilerParams(
        needs_layout_passes=False, use_tc_tiling_on_sc=True),
)
def _interp_sc(x_hbm, y_hbm, in0, in1, out0, out1, sg0, sg1, ss0, ss1):
    wid = lax.axis_index("s") * 2 + lax.axis_index("c")
    row0 = wid * _RPW
    _CPR = _W // _CC
    _NCHUNK = (_RPW // _CR) * _CPR

    evens = lax.iota(jnp.int32, _L) * 2
    rowsel = [jnp.full((_L,), s, jnp.int32) for s in range(_CR)]

    def src_slice(c):
        r = row0 + (c // _CPR) * _CR
        col = (c % _CPR) * _CC
        return x_hbm.at[pl.ds(r, _CR), pl.ds(col, _CC)]

    def dst_slice(c):
        r = row0 + (c // _CPR) * _CR
        col = (c % _CPR) * _CC
        return y_hbm.at[pl.ds(r, _CR), pl.ds(2 * col, 2 * _CC)]

    def make_ibody(in_v, out_v):
        def ibody(i, carry):
            for s in range(_CR):
                xv = in_v[s, pl.ds(i * _L, _L)]
                plsc.store_scatter(
                    out_v, [rowsel[s], evens + i * (2 * _L)], xv)
            return carry
        return ibody

    bufs = ((in0, out0, sg0, ss0), (in1, out1, sg1, ss1))

    pltpu.async_copy(src_slice(0), in0, sg0)
    pltpu.async_copy(src_slice(1), in1, sg1)

    zeros = jnp.zeros((_L,), jnp.float32)

    def zbody(i, carry):
        for s in range(_CR):
            out0[s, pl.ds(i * _L, _L)] = zeros
            out1[s, pl.ds(i * _L, _L)] = zeros
        return carry

    lax.fori_loop(0, (2 * _CC) // _L, zbody, 0)

    def cbody(c2, carry):
        for b, (in_v, out_v, sg, ss) in enumerate(bufs):
            cc = c2 * 2 + b
            pltpu.make_async_copy(src_slice(cc), in_v, sg).wait()
            @pl.when(cc >= 2)
            def _():
                pltpu.make_async_copy(out_v, dst_slice(cc), ss).wait()
            lax.fori_loop(0, _CC // _L, make_ibody(in_v, out_v), 0)
            pltpu.async_copy(out_v, dst_slice(cc), ss)
            @pl.when(cc + 2 < _NCHUNK)
            def _():
                pltpu.async_copy(src_slice(cc + 2), in_v, sg)
        return carry

    lax.fori_loop(0, _NCHUNK // 2, cbody, 0)

    pltpu.make_async_copy(out0, dst_slice(_NCHUNK - 2), ss0).wait()
    pltpu.make_async_copy(out1, dst_slice(_NCHUNK - 1), ss1).wait()


_TBR = 512
_TBC = 1024


def _tc_body(x_ref, y_ref):
    lanes = lax.broadcasted_iota(jnp.int32, (128, 256), 0)
    cols = lax.broadcasted_iota(jnp.int32, (128, 256), 1)
    e16 = jnp.where(cols == 2 * lanes, 1.0, 0.0).astype(jnp.bfloat16)
    xb = x_ref[...]
    a = xb.astype(jnp.bfloat16)
    r = xb - a.astype(jnp.float32)
    b = r.astype(jnp.bfloat16)
    c = (r - b.astype(jnp.float32)).astype(jnp.bfloat16)
    for k in range(_TBC // 128):
        sl = slice(128 * k, 128 * (k + 1))
        acc = jax.lax.dot(a[:, sl], e16, preferred_element_type=jnp.float32)
        acc = acc + jax.lax.dot(
            b[:, sl], e16, preferred_element_type=jnp.float32)
        acc = acc + jax.lax.dot(
            c[:, sl], e16, preferred_element_type=jnp.float32)
        y_ref[:, 256 * k:256 * (k + 1)] = acc


def _interp_tc(x2):
    rows = _R - _RSC
    off = _RSC // _TBR
    return pl.pallas_call(
        _tc_body,
        out_shape=jax.ShapeDtypeStruct((rows, 2 * _W), jnp.float32),
        grid=(rows // _TBR, _W // _TBC),
        in_specs=[pl.BlockSpec((_TBR, _TBC), lambda i, j: (i + off, j))],
        out_specs=pl.BlockSpec((_TBR, 2 * _TBC), lambda i, j: (i, j)),
    )(x2)


def kernel(x):
    x2 = x.reshape(_R, _W)
    y_sc = _interp_sc(x2)
    y_tc = _interp_tc(x2)
    y = jnp.concatenate([y_sc, y_tc], axis=0)
    return y.reshape(_B, _S, 2 * _W)
